# baseline (speedup 1.0000x reference)
import functools

import jax
import jax.numpy as jnp
from jax import lax
from jax.experimental import pallas as pl
from jax.experimental.pallas import tpu as pltpu
from jax.experimental.pallas import tpu_sc as plsc

VOCAB = 1000000
EMBED = 64
BATCH = 16384
CTX = 20

NC = 2
NS = 16
NW = NC * NS
NB = BATCH // NW
SB = 32
NCH = NB // SB
GATHER = 128
CG = SB * CTX // GATHER
TG = NB // GATHER
LANES = 16
NV = EMBED // LANES


def _make_cbow():
    mesh = plsc.VectorSubcoreMesh(core_axis_name="c", subcore_axis_name="s")

    @functools.partial(
        pl.kernel,
        mesh=mesh,
        out_type=jax.ShapeDtypeStruct((BATCH,), jnp.float32),
        compiler_params=pltpu.CompilerParams(
            needs_layout_passes=False, use_tc_tiling_on_sc=False),
        scratch_types=[
            pltpu.VMEM((NB * CTX // GATHER, GATHER), jnp.int32),
            pltpu.VMEM((TG, GATHER), jnp.int32),
            pltpu.VMEM((SB * CTX, EMBED), jnp.float32),
            pltpu.VMEM((NB, EMBED), jnp.float32),
            pltpu.VMEM((NB,), jnp.float32),
            pltpu.SemaphoreType.DMA,
        ],
    )
    def cbow(w_in_hbm, w_out_hbm, ctx_hbm, tgt_hbm, out_hbm,
             cidx_v, tidx_v, crows_v, trows_v, out_v, sem):
        wid = lax.axis_index("s") * NC + lax.axis_index("c")
        base = wid * NB
        nrow_c = NB * CTX // GATHER

        for t in range(NB // LANES):
            out_v[pl.ds(t * LANES, LANES)] = jnp.zeros((LANES,), jnp.float32)

        pltpu.sync_copy(ctx_hbm.at[pl.ds(wid * nrow_c, nrow_c)], cidx_v)
        pltpu.sync_copy(tgt_hbm.at[pl.ds(wid * TG, TG)], tidx_v)

        tcps = [pltpu.async_copy(w_out_hbm.at[tidx_v.at[j]],
                                 trows_v.at[pl.ds(j * GATHER, GATHER)], sem)
                for j in range(TG)]
        for cp in tcps:
            cp.wait()

        def chunk_body(g, _):
            ccps = [pltpu.async_copy(w_in_hbm.at[cidx_v.at[g * CG + j]],
                                     crows_v.at[pl.ds(j * GATHER, GATHER)], sem)
                    for j in range(CG)]
            for cp in ccps:
                cp.wait()

            def row_body(i, _):
                b = g * SB + i
                r0 = i * CTX
                accs = [jnp.zeros((LANES,), jnp.float32) for _ in range(NV)]
                for l in range(CTX):
                    for k in range(NV):
                        accs[k] = accs[k] + crows_v[r0 + l, pl.ds(k * LANES, LANES)]
                p = jnp.zeros((LANES,), jnp.float32)
                for k in range(NV):
                    p = p + accs[k] * trows_v[b, pl.ds(k * LANES, LANES)]
                plsc.addupdate_scatter(
                    out_v, [jnp.full((LANES,), b, jnp.int32)], p * (1.0 / CTX))
                return 0

            lax.fori_loop(0, SB, row_body, 0)
            return 0

        lax.fori_loop(0, NCH, chunk_body, 0)
        pltpu.sync_copy(out_v, out_hbm.at[pl.ds(base, NB)])

    return cbow


_cbow = _make_cbow()


def kernel(W_in, W_out, context, target):
    ctx2d = context.astype(jnp.int32).reshape(BATCH * CTX // GATHER, GATHER)
    tgt2d = target.astype(jnp.int32).reshape(BATCH // GATHER, GATHER)
    return _cbow(W_in, W_out, ctx2d, tgt2d)


# speedup vs baseline: 1.4809x; 1.4809x over previous
import functools

import jax
import jax.numpy as jnp
from jax import lax
from jax.experimental import pallas as pl
from jax.experimental.pallas import tpu as pltpu
from jax.experimental.pallas import tpu_sc as plsc

VOCAB = 1000000
EMBED = 64
BATCH = 16384
CTX = 20

NC = 2
NS = 16
NW = NC * NS
NB = BATCH // NW
SB = 32
NCH = NB // SB
LANES = 16
NV = EMBED // LANES
IW = 128
CR = SB * CTX // 2
TR = NB // 2


def _make_cbow():
    mesh = plsc.VectorSubcoreMesh(core_axis_name="c", subcore_axis_name="s")

    @functools.partial(
        pl.kernel,
        mesh=mesh,
        out_type=jax.ShapeDtypeStruct((BATCH,), jnp.float32),
        compiler_params=pltpu.CompilerParams(
            needs_layout_passes=False, use_tc_tiling_on_sc=True),
        scratch_types=[
            pltpu.VMEM((NB * CTX // IW, IW), jnp.int32),
            pltpu.VMEM((NB // IW, IW), jnp.int32),
            pltpu.VMEM((CR, IW), jnp.float32),
            pltpu.VMEM((TR, IW), jnp.float32),
            pltpu.VMEM((NB,), jnp.float32),
            pltpu.SemaphoreType.DMA,
            pltpu.SemaphoreType.DMA,
        ],
    )
    def cbow(w_in_hbm, w_out_hbm, ctx_hbm, tgt_hbm, dummy_hbm, out_hbm,
             cidx_v, tidx_v, crows_v, trows_v, out_v, sem, tsem):
        wid = lax.axis_index("s") * NC + lax.axis_index("c")

        for t in range(NB // LANES):
            out_v[pl.ds(t * LANES, LANES)] = jnp.zeros((LANES,), jnp.float32)

        pltpu.sync_copy(ctx_hbm.at[wid], cidx_v)
        pltpu.sync_copy(tgt_hbm.at[wid], tidx_v)

        def tgt_fetch(q, _):
            j0 = q * LANES
            idxv = tidx_v[j0 // IW, pl.ds(j0 % IW, LANES)]
            for k in range(LANES):
                pltpu.async_copy(
                    w_out_hbm.at[idxv[k]],
                    trows_v.at[q * (LANES // 2) + k // 2,
                               pl.ds((k % 2) * EMBED, EMBED)],
                    tsem)
            return 0

        lax.fori_loop(0, NB // LANES, tgt_fetch, 0)

        def chunk_body(g, _):
            def ctx_fetch(q, _):
                jj = g * (SB * CTX) + q * LANES
                idxv = cidx_v[jj // IW, pl.ds(jj % IW, LANES)]
                for k in range(LANES):
                    pltpu.async_copy(
                        w_in_hbm.at[idxv[k]],
                        crows_v.at[q * (LANES // 2) + k // 2,
                                   pl.ds((k % 2) * EMBED, EMBED)],
                        sem)
                return 0

            lax.fori_loop(0, SB * CTX // LANES, ctx_fetch, 0)
            pltpu.make_async_copy(dummy_hbm, crows_v, sem).wait()

            def row_body(i, _):
                b = g * SB + i
                accs = [jnp.zeros((LANES,), jnp.float32) for _ in range(NV)]
                for l in range(CTX):
                    r2 = i * (CTX // 2) + l // 2
                    c0 = (l % 2) * EMBED
                    for k in range(NV):
                        accs[k] = accs[k] + crows_v[r2, pl.ds(c0 + k * LANES, LANES)]
                b2 = g * (SB // 2) + i // 2
                t0 = (i % 2) * EMBED
                p = jnp.zeros((LANES,), jnp.float32)
                for k in range(NV):
                    p = p + accs[k] * trows_v[b2, pl.ds(t0 + k * LANES, LANES)]
                plsc.addupdate_scatter(
                    out_v, [jnp.full((LANES,), b, jnp.int32)], p * (1.0 / CTX))
                return 0

            lax.fori_loop(0, SB, row_body, 0)
            return 0

        pltpu.make_async_copy(dummy_hbm.at[pl.ds(0, TR)], trows_v, tsem).wait()
        lax.fori_loop(0, NCH, chunk_body, 0)
        pltpu.sync_copy(out_v, out_hbm.at[pl.ds(wid * NB, NB)])

    return cbow


_cbow = _make_cbow()


def kernel(W_in, W_out, context, target):
    ctx3d = context.astype(jnp.int32).reshape(NW, NB * CTX // IW, IW)
    tgt3d = target.astype(jnp.int32).reshape(NW, NB // IW, IW)
    dummy = jnp.zeros((CR, IW), jnp.float32)
    return _cbow(W_in, W_out, ctx3d, tgt3d, dummy)


# speedup vs baseline: 1.5028x; 1.0148x over previous
import functools

import jax
import jax.numpy as jnp
from jax import lax
from jax.experimental import pallas as pl
from jax.experimental.pallas import tpu as pltpu
from jax.experimental.pallas import tpu_sc as plsc

VOCAB = 1000000
EMBED = 64
BATCH = 16384
CTX = 20

NC = 2
NS = 16
NW = NC * NS
NB = BATCH // NW
SB = 16
NCH = NB // SB
LANES = 16
NV = EMBED // LANES
IW = 128
CR = SB * CTX // 2
TR = NB // 2


def _make_cbow():
    mesh = plsc.VectorSubcoreMesh(core_axis_name="c", subcore_axis_name="s")

    @functools.partial(
        pl.kernel,
        mesh=mesh,
        out_type=jax.ShapeDtypeStruct((BATCH,), jnp.float32),
        compiler_params=pltpu.CompilerParams(
            needs_layout_passes=False, use_tc_tiling_on_sc=True),
        scratch_types=[
            pltpu.VMEM((NB * CTX // IW, IW), jnp.int32),
            pltpu.VMEM((NB // IW, IW), jnp.int32),
            pltpu.VMEM((2, CR, IW), jnp.float32),
            pltpu.VMEM((TR, IW), jnp.float32),
            pltpu.VMEM((NB,), jnp.float32),
            pltpu.SemaphoreType.DMA,
            pltpu.SemaphoreType.DMA,
            pltpu.SemaphoreType.DMA,
        ],
    )
    def cbow(w_in_hbm, w_out_hbm, ctx_hbm, tgt_hbm, dummy_hbm, out_hbm,
             cidx_v, tidx_v, crows_v, trows_v, out_v, sem0, sem1, tsem):
        wid = lax.axis_index("s") * NC + lax.axis_index("c")
        sems = (sem0, sem1)

        for t in range(NB // LANES):
            out_v[pl.ds(t * LANES, LANES)] = jnp.zeros((LANES,), jnp.float32)

        pltpu.sync_copy(ctx_hbm.at[wid], cidx_v)
        pltpu.sync_copy(tgt_hbm.at[wid], tidx_v)

        def tgt_fetch(q, _):
            j0 = q * LANES
            idxv = tidx_v[j0 // IW, pl.ds(j0 % IW, LANES)]
            for k in range(LANES):
                pltpu.async_copy(
                    w_out_hbm.at[idxv[k]],
                    trows_v.at[q * (LANES // 2) + k // 2,
                               pl.ds((k % 2) * EMBED, EMBED)],
                    tsem)
            return 0

        lax.fori_loop(0, NB // LANES, tgt_fetch, 0)

        def fire_chunk(c, buf, sem):
            def ctx_fetch(q, _):
                jj = c * (SB * CTX) + q * LANES
                idxv = cidx_v[jj // IW, pl.ds(jj % IW, LANES)]
                for k in range(LANES):
                    pltpu.async_copy(
                        w_in_hbm.at[idxv[k]],
                        crows_v.at[buf, q * (LANES // 2) + k // 2,
                                   pl.ds((k % 2) * EMBED, EMBED)],
                        sem)
                return 0

            lax.fori_loop(0, SB * CTX // LANES, ctx_fetch, 0)

        def compute_chunk(c, buf):
            def row_body(i, _):
                b = c * SB + i
                accs = [jnp.zeros((LANES,), jnp.float32) for _ in range(NV)]
                for l in range(CTX):
                    r2 = i * (CTX // 2) + l // 2
                    c0 = (l % 2) * EMBED
                    for k in range(NV):
                        accs[k] = accs[k] + crows_v[buf, r2,
                                                    pl.ds(c0 + k * LANES, LANES)]
                b2 = b // 2
                t0 = (b % 2) * EMBED
                p = jnp.zeros((LANES,), jnp.float32)
                for k in range(NV):
                    p = p + accs[k] * trows_v[b2, pl.ds(t0 + k * LANES, LANES)]
                plsc.addupdate_scatter(
                    out_v, [jnp.full((LANES,), b, jnp.int32)], p * (1.0 / CTX))
                return 0

            lax.fori_loop(0, SB, row_body, 0)

        def drain(buf, sem):
            pltpu.make_async_copy(
                dummy_hbm.at[pl.ds(0, CR)], crows_v.at[buf], sem).wait()

        pltpu.make_async_copy(dummy_hbm, trows_v, tsem).wait()

        fire_chunk(0, 0, sems[0])
        fire_chunk(1, 1, sems[1])

        def pair_body(t, _):
            g = t * 2
            for b in range(2):
                drain(b, sems[b])
                compute_chunk(g + b, b)
                fire_chunk(g + b + 2, b, sems[b])
            return 0

        lax.fori_loop(0, NCH // 2 - 1, pair_body, 0)
        for b in range(2):
            drain(b, sems[b])
            compute_chunk(NCH - 2 + b, b)

        pltpu.sync_copy(out_v, out_hbm.at[pl.ds(wid * NB, NB)])

    return cbow


_cbow = _make_cbow()


def kernel(W_in, W_out, context, target):
    ctx3d = context.astype(jnp.int32).reshape(NW, NB * CTX // IW, IW)
    tgt3d = target.astype(jnp.int32).reshape(NW, NB // IW, IW)
    dummy = jnp.zeros((TR, IW), jnp.float32)
    return _cbow(W_in, W_out, ctx3d, tgt3d, dummy)


# speedup vs baseline: 2.2149x; 1.4739x over previous
import functools

import jax
import jax.numpy as jnp
from jax import lax
from jax.experimental import pallas as pl
from jax.experimental.pallas import tpu as pltpu
from jax.experimental.pallas import tpu_sc as plsc

VOCAB = 1000000
EMBED = 64
BATCH = 16384
CTX = 20

NC = 2
NS = 16
NW = NC * NS
NB = BATCH // NW
SB = 16
NCH = NB // SB
LANES = 16
NV = EMBED // LANES
IW = 128
CR = SB * CTX // 2
TR = NB // 2
TBK = 4
NTR = NB // (4 * TBK)

_MESH = plsc.VectorSubcoreMesh(core_axis_name="c", subcore_axis_name="s")
_PARAMS = pltpu.CompilerParams(
    needs_layout_passes=False, use_tc_tiling_on_sc=True)


def _make_tgt():
    @functools.partial(
        pl.kernel,
        mesh=_MESH,
        out_type=jax.ShapeDtypeStruct((NW, TR, IW), jnp.float32),
        compiler_params=_PARAMS,
        scratch_types=[
            pltpu.VMEM((NB // IW, IW), jnp.int32),
            pltpu.VMEM((TR, IW), jnp.float32),
            pltpu.VMEM((2, TBK, EMBED, IW), jnp.float32),
            pltpu.SemaphoreType.DMA,
        ],
    )
    def tgt_kernel(w_out_t_hbm, tgt_hbm, dummy_t_hbm, out_hbm,
                   tidx_v, trows_v, tblk_v, tsem):
        wid = lax.axis_index("s") * NC + lax.axis_index("c")
        pltpu.sync_copy(tgt_hbm.at[wid], tidx_v)

        rows16 = [jnp.arange(16, dtype=jnp.int32) + 16 * j for j in range(NV)]

        def load_tidx(t):
            j0 = t * 16
            return tidx_v[j0 // IW, pl.ds(j0 % IW, 16)]

        def fire_quarter(idxs, ring):
            for k in range(TBK):
                qoff = (idxs[k] // IW) * IW
                pltpu.async_copy(
                    w_out_t_hbm.at[:, pl.ds(pl.multiple_of(qoff, IW), IW)],
                    tblk_v.at[ring, k], tsem)

        def extract_quarter(t, h, idxs):
            ring = h % 2
            for k in range(TBK):
                r = idxs[k] % IW
                lanes = jnp.full((16,), r, jnp.int32)
                b2 = t * 8 + h * 2 + k // 2
                t0 = (k % 2) * EMBED
                for j in range(NV):
                    col = plsc.load_gather(
                        tblk_v.at[ring, k], [rows16[j], lanes])
                    trows_v[b2, pl.ds(t0 + j * LANES, LANES)] = col

        idxv0 = load_tidx(0)
        fire_quarter([idxv0[k] for k in range(TBK)], 0)

        def tgt_round(t, _):
            idxv = load_tidx(t)
            idxn = load_tidx(jnp.minimum(t + 1, NTR - 1))
            for h in range(4):
                if h < 3:
                    nxt = [idxv[4 * (h + 1) + k] for k in range(TBK)]
                else:
                    nxt = [idxn[k] for k in range(TBK)]
                fire_quarter(nxt, (h + 1) % 2)
                pltpu.make_async_copy(
                    dummy_t_hbm, tblk_v.at[h % 2], tsem).wait()
                extract_quarter(t, h, [idxv[4 * h + k] for k in range(TBK)])
            return 0

        lax.fori_loop(0, NTR, tgt_round, 0)
        pltpu.make_async_copy(dummy_t_hbm, tblk_v.at[0], tsem).wait()
        pltpu.sync_copy(trows_v, out_hbm.at[wid])

    return tgt_kernel


def _make_main():
    @functools.partial(
        pl.kernel,
        mesh=_MESH,
        out_type=jax.ShapeDtypeStruct((BATCH,), jnp.float32),
        compiler_params=_PARAMS,
        scratch_types=[
            pltpu.VMEM((NB * CTX // IW, IW), jnp.int32),
            pltpu.VMEM((2, CR, IW), jnp.float32),
            pltpu.VMEM((TR, IW), jnp.float32),
            pltpu.VMEM((NB,), jnp.float32),
            pltpu.SemaphoreType.DMA,
            pltpu.SemaphoreType.DMA,
        ],
    )
    def main_kernel(w_in_hbm, ctx_hbm, trows_hbm, dummy_c_hbm, out_hbm,
                    cidx_v, crows_v, trows_v, out_v, sem0, sem1):
        wid = lax.axis_index("s") * NC + lax.axis_index("c")
        sems = (sem0, sem1)

        for t in range(NB // LANES):
            out_v[pl.ds(t * LANES, LANES)] = jnp.zeros((LANES,), jnp.float32)

        pltpu.sync_copy(ctx_hbm.at[wid], cidx_v)
        pltpu.sync_copy(trows_hbm.at[wid], trows_v)

        def fire_chunk(c, buf, sem):
            def ctx_fetch(q, _):
                jj = c * (SB * CTX) + q * LANES
                idxv = cidx_v[jj // IW, pl.ds(jj % IW, LANES)]
                for k in range(LANES):
                    pltpu.async_copy(
                        w_in_hbm.at[idxv[k]],
                        crows_v.at[buf, q * (LANES // 2) + k // 2,
                                   pl.ds((k % 2) * EMBED, EMBED)],
                        sem)
                return 0

            lax.fori_loop(0, SB * CTX // LANES, ctx_fetch, 0)

        def compute_chunk(c, buf):
            def row_body(i, _):
                b = c * SB + i
                accs = [jnp.zeros((LANES,), jnp.float32) for _ in range(NV)]
                for l in range(CTX):
                    r2 = i * (CTX // 2) + l // 2
                    c0 = (l % 2) * EMBED
                    for k in range(NV):
                        accs[k] = accs[k] + crows_v[buf, r2,
                                                    pl.ds(c0 + k * LANES, LANES)]
                b2 = b // 2
                t0 = (b % 2) * EMBED
                p = jnp.zeros((LANES,), jnp.float32)
                for k in range(NV):
                    p = p + accs[k] * trows_v[b2, pl.ds(t0 + k * LANES, LANES)]
                plsc.addupdate_scatter(
                    out_v, [jnp.full((LANES,), b, jnp.int32)], p * (1.0 / CTX))
                return 0

            lax.fori_loop(0, SB, row_body, 0)

        def drain(buf, sem):
            pltpu.make_async_copy(dummy_c_hbm, crows_v.at[buf], sem).wait()

        fire_chunk(0, 0, sems[0])
        fire_chunk(1, 1, sems[1])

        def pair_body(t, _):
            g = t * 2
            for b in range(2):
                drain(b, sems[b])
                compute_chunk(g + b, b)
                fire_chunk(g + b + 2, b, sems[b])
            return 0

        lax.fori_loop(0, NCH // 2 - 1, pair_body, 0)
        for b in range(2):
            drain(b, sems[b])
            compute_chunk(NCH - 2 + b, b)

        pltpu.sync_copy(out_v, out_hbm.at[pl.ds(wid * NB, NB)])

    return main_kernel


_tgt = _make_tgt()
_main = _make_main()


def kernel(W_in, W_out, context, target):
    wout_t = jnp.swapaxes(W_out, 0, 1)
    ctx3d = context.astype(jnp.int32).reshape(NW, NB * CTX // IW, IW)
    tgt3d = target.astype(jnp.int32).reshape(NW, NB // IW, IW)
    dummy_c = jnp.zeros((CR, IW), jnp.float32)
    dummy_t = jnp.zeros((TBK, EMBED, IW), jnp.float32)
    trows = _tgt(wout_t, tgt3d, dummy_t)
    return _main(W_in, ctx3d, trows, dummy_c)


# speedup vs baseline: 2.2172x; 1.0010x over previous
import functools

import jax
import jax.numpy as jnp
from jax import lax
from jax.experimental import pallas as pl
from jax.experimental.pallas import tpu as pltpu
from jax.experimental.pallas import tpu_sc as plsc

VOCAB = 1000000
EMBED = 64
BATCH = 16384
CTX = 20

NC = 2
NS = 16
NW = NC * NS
NB = BATCH // NW
SB = 16
NCH = NB // SB
LANES = 16
NV = EMBED // LANES
IW = 128
CR = SB * CTX // 2
TR = NB // 2
TBK = 4
NTR = NB // (4 * TBK)

_MESH = plsc.VectorSubcoreMesh(core_axis_name="c", subcore_axis_name="s")
_PARAMS = pltpu.CompilerParams(
    needs_layout_passes=False, use_tc_tiling_on_sc=True)


def _make_tgt():
    @functools.partial(
        pl.kernel,
        mesh=_MESH,
        out_type=jax.ShapeDtypeStruct((NW, TR, IW), jnp.float32),
        compiler_params=_PARAMS,
        scratch_types=[
            pltpu.VMEM((NB // IW, IW), jnp.int32),
            pltpu.VMEM((TR, IW), jnp.float32),
            pltpu.VMEM((2, TBK, EMBED, IW), jnp.float32),
            pltpu.SemaphoreType.DMA,
        ],
    )
    def tgt_kernel(w_out_t_hbm, tgt_hbm, dummy_t_hbm, out_hbm,
                   tidx_v, trows_v, tblk_v, tsem):
        wid = lax.axis_index("s") * NC + lax.axis_index("c")
        pltpu.sync_copy(tgt_hbm.at[wid], tidx_v)

        rows16 = [jnp.arange(16, dtype=jnp.int32) + 16 * j for j in range(NV)]

        def load_tidx(t):
            j0 = t * 16
            return tidx_v[j0 // IW, pl.ds(j0 % IW, 16)]

        def fire_quarter(idxs, ring):
            for k in range(TBK):
                qoff = (idxs[k] // IW) * IW
                pltpu.async_copy(
                    w_out_t_hbm.at[:, pl.ds(pl.multiple_of(qoff, IW), IW)],
                    tblk_v.at[ring, k], tsem)

        def extract_quarter(t, h, idxs):
            ring = h % 2
            for k in range(TBK):
                r = idxs[k] % IW
                lanes = jnp.full((16,), r, jnp.int32)
                b2 = t * 8 + h * 2 + k // 2
                t0 = (k % 2) * EMBED
                for j in range(NV):
                    col = plsc.load_gather(
                        tblk_v.at[ring, k], [rows16[j], lanes])
                    trows_v[b2, pl.ds(t0 + j * LANES, LANES)] = col

        idxv0 = load_tidx(0)
        fire_quarter([idxv0[k] for k in range(TBK)], 0)

        def tgt_round(t, _):
            idxv = load_tidx(t)
            idxn = load_tidx(jnp.minimum(t + 1, NTR - 1))
            for h in range(4):
                if h < 3:
                    nxt = [idxv[4 * (h + 1) + k] for k in range(TBK)]
                else:
                    nxt = [idxn[k] for k in range(TBK)]
                fire_quarter(nxt, (h + 1) % 2)
                pltpu.make_async_copy(
                    dummy_t_hbm, tblk_v.at[h % 2], tsem).wait()
                extract_quarter(t, h, [idxv[4 * h + k] for k in range(TBK)])
            return 0

        lax.fori_loop(0, NTR, tgt_round, 0)
        pltpu.make_async_copy(dummy_t_hbm, tblk_v.at[0], tsem).wait()
        pltpu.sync_copy(trows_v, out_hbm.at[wid])

    return tgt_kernel


def _make_main():
    @functools.partial(
        pl.kernel,
        mesh=_MESH,
        out_type=jax.ShapeDtypeStruct((BATCH,), jnp.float32),
        compiler_params=_PARAMS,
        scratch_types=[
            pltpu.VMEM((NB * CTX // IW, IW), jnp.int32),
            pltpu.VMEM((2, CR, IW), jnp.float32),
            pltpu.VMEM((TR, IW), jnp.float32),
            pltpu.VMEM((NB,), jnp.float32),
            pltpu.SemaphoreType.DMA,
            pltpu.SemaphoreType.DMA,
        ],
    )
    def main_kernel(w_in_hbm, ctx_hbm, trows_hbm, dummy_c_hbm, out_hbm,
                    cidx_v, crows_v, trows_v, out_v, sem0, sem1):
        wid = lax.axis_index("s") * NC + lax.axis_index("c")
        sems = (sem0, sem1)

        for t in range(NB // LANES):
            out_v[pl.ds(t * LANES, LANES)] = jnp.zeros((LANES,), jnp.float32)

        pltpu.sync_copy(ctx_hbm.at[wid], cidx_v)
        pltpu.sync_copy(trows_hbm.at[wid], trows_v)

        def fire_chunk(c, buf, sem):
            def ctx_fetch(q, _):
                jj = c * (SB * CTX) + q * LANES
                idxv = cidx_v[jj // IW, pl.ds(jj % IW, LANES)]
                for k in range(LANES):
                    pltpu.async_copy(
                        w_in_hbm.at[idxv[k]],
                        crows_v.at[buf, q * (LANES // 2) + k // 2,
                                   pl.ds((k % 2) * EMBED, EMBED)],
                        sem)
                return 0

            lax.fori_loop(0, SB * CTX // LANES, ctx_fetch, 0)

        def compute_chunk(c, buf):
            def row_body(i, _):
                b = c * SB + i
                accs = [jnp.zeros((LANES,), jnp.float32) for _ in range(NV)]
                for l in range(CTX):
                    r2 = i * (CTX // 2) + l // 2
                    c0 = (l % 2) * EMBED
                    for k in range(NV):
                        accs[k] = accs[k] + crows_v[buf, r2,
                                                    pl.ds(c0 + k * LANES, LANES)]
                b2 = b // 2
                t0 = (b % 2) * EMBED
                p = jnp.zeros((LANES,), jnp.float32)
                for k in range(NV):
                    p = p + accs[k] * trows_v[b2, pl.ds(t0 + k * LANES, LANES)]
                plsc.addupdate_scatter(
                    out_v, [jnp.full((LANES,), b, jnp.int32)], p * (1.0 / CTX))
                return 0

            lax.fori_loop(0, SB, row_body, 0)

        def drain(buf, sem):
            pltpu.make_async_copy(dummy_c_hbm, crows_v.at[buf], sem).wait()

        fire_chunk(0, 0, sems[0])
        fire_chunk(1, 1, sems[1])

        def pair_body(t, _):
            g = t * 2
            for b in range(2):
                drain(b, sems[b])
                compute_chunk(g + b, b)
                fire_chunk(g + b + 2, b, sems[b])
            return 0

        lax.fori_loop(0, NCH // 2 - 1, pair_body, 0)
        for b in range(2):
            drain(b, sems[b])
            compute_chunk(NCH - 2 + b, b)

        pltpu.sync_copy(out_v, out_hbm.at[pl.ds(wid * NB, NB)])

    return main_kernel


_tgt = _make_tgt()
_main = _make_main()

TRB = 4096


def _tr_body(x_ref, o_ref):
    o_ref[...] = x_ref[...].T


_transpose = pl.pallas_call(
    _tr_body,
    grid=((VOCAB + TRB - 1) // TRB,),
    in_specs=[pl.BlockSpec((EMBED, TRB), lambda i: (0, i))],
    out_specs=pl.BlockSpec((TRB, EMBED), lambda i: (i, 0)),
    out_shape=jax.ShapeDtypeStruct((VOCAB, EMBED), jnp.float32),
)


def kernel(W_in, W_out, context, target):
    wout_t = jnp.swapaxes(W_out, 0, 1)
    ctx3d = context.astype(jnp.int32).reshape(NW, NB * CTX // IW, IW)
    tgt3d = target.astype(jnp.int32).reshape(NW, NB // IW, IW)
    dummy_c = jnp.zeros((CR, IW), jnp.float32)
    dummy_t = jnp.zeros((TBK, EMBED, IW), jnp.float32)
    win_t = jnp.swapaxes(W_in, 0, 1)
    trows = _tgt(wout_t, tgt3d, dummy_t)
    win_rm = _transpose(win_t)
    return _main(win_rm, ctx3d, trows, dummy_c)
